# Initial kernel scaffold; baseline (speedup 1.0000x reference)
import jax
import jax.numpy as jnp
from jax.experimental import pallas as pl


def kernel(firings):
    raise NotImplementedError("write your pallas kernel here")


# speedup vs baseline: 1.0095x; 1.0095x over previous
import jax
import jax.numpy as jnp
from jax import lax
from jax.experimental import pallas as pl
from jax.experimental.pallas import tpu as pltpu
from jax.experimental.pallas import tpu_sc as plsc

_LANES = 16
_NUM_CORES = 2
_NUM_SUBCORES = 16
_NW = _NUM_CORES * _NUM_SUBCORES

_B, _R, _F = 4, 4096, 256
_OF = _F // 2
_IN_TOTAL = _B * _R * _F
_OUT_TOTAL = _B * _R * _OF
_IN_PER_W = _IN_TOTAL // _NW
_OUT_PER_W = _OUT_TOTAL // _NW
_NCHUNK = 4
_IN_CHUNK = _IN_PER_W // _NCHUNK
_OUT_CHUNK = _OUT_PER_W // _NCHUNK
_VECS = _OUT_CHUNK // _LANES


def _sel_body(x_hbm, out_hbm, in_v, out_v):
    wid = lax.axis_index("s") * _NUM_CORES + lax.axis_index("c")
    in_base = wid * _IN_PER_W
    out_base = wid * _OUT_PER_W
    iota2 = lax.iota(jnp.int32, _LANES) * 2

    for t in range(_NCHUNK):
        pltpu.sync_copy(x_hbm.at[pl.ds(in_base + t * _IN_CHUNK, _IN_CHUNK)], in_v)

        def body(v, carry):
            idx = iota2 + v * (2 * _LANES)
            out_v[pl.ds(v * _LANES, _LANES)] = plsc.load_gather(in_v, [idx])
            return carry

        lax.fori_loop(0, _VECS, body, 0)
        pltpu.sync_copy(out_v, out_hbm.at[pl.ds(out_base + t * _OUT_CHUNK, _OUT_CHUNK)])


_sel = pl.kernel(
    _sel_body,
    out_type=jax.ShapeDtypeStruct((_OUT_TOTAL,), jnp.float32),
    mesh=plsc.VectorSubcoreMesh(
        core_axis_name="c",
        subcore_axis_name="s",
        num_cores=_NUM_CORES,
        num_subcores=_NUM_SUBCORES,
    ),
    scratch_types=[
        pltpu.VMEM((_IN_CHUNK,), jnp.float32),
        pltpu.VMEM((_OUT_CHUNK,), jnp.float32),
    ],
    compiler_params=pltpu.CompilerParams(needs_layout_passes=False),
)


def kernel(firings):
    out = _sel(firings.reshape(_IN_TOTAL))
    return out.reshape(_B, _R, _OF)


# speedup vs baseline: 1.1880x; 1.1768x over previous
import jax
import jax.numpy as jnp
from jax import lax
from jax.experimental import pallas as pl
from jax.experimental.pallas import tpu as pltpu
from jax.experimental.pallas import tpu_sc as plsc

_LANES = 16
_NUM_CORES = 2
_NUM_SUBCORES = 16
_NW = _NUM_CORES * _NUM_SUBCORES

_B, _R, _F = 4, 4096, 256
_OF = _F // 2
_IN_TOTAL = _B * _R * _F
_OUT_TOTAL = _B * _R * _OF
_IN_PER_W = _IN_TOTAL // _NW
_OUT_PER_W = _OUT_TOTAL // _NW
_NCHUNK = 4
_IN_CHUNK = _IN_PER_W // _NCHUNK
_OUT_CHUNK = _OUT_PER_W // _NCHUNK
_VECS = _OUT_CHUNK // _LANES


_UNROLL = 8


def _sel_body(x_hbm, out_hbm, in_v0, in_v1, out_v0, out_v1, in_sem, out_sem):
    in_bufs = (in_v0, in_v1)
    out_bufs = (out_v0, out_v1)
    wid = lax.axis_index("s") * _NUM_CORES + lax.axis_index("c")
    in_base = wid * _IN_PER_W
    out_base = wid * _OUT_PER_W
    iota2 = lax.iota(jnp.int32, _LANES) * 2

    def start_in(t):
        return pltpu.async_copy(
            x_hbm.at[pl.ds(in_base + t * _IN_CHUNK, _IN_CHUNK)],
            in_bufs[t % 2], in_sem)

    def start_out(t):
        return pltpu.async_copy(
            out_bufs[t % 2],
            out_hbm.at[pl.ds(out_base + t * _OUT_CHUNK, _OUT_CHUNK)], out_sem)

    copies_in = [start_in(0)]
    copies_out = []
    for t in range(_NCHUNK):
        if t + 1 < _NCHUNK:
            copies_in.append(start_in(t + 1))
        copies_in[t].wait()
        if t >= 2:
            copies_out[t - 2].wait()
        src = in_bufs[t % 2]
        dst = out_bufs[t % 2]

        def body(v, idx):
            for u in range(_UNROLL):
                dst[pl.ds((v * _UNROLL + u) * _LANES, _LANES)] = (
                    plsc.load_gather(src, [idx + u * (2 * _LANES)]))
            return idx + _UNROLL * 2 * _LANES

        lax.fori_loop(0, _VECS // _UNROLL, body, iota2)
        copies_out.append(start_out(t))
    copies_out[-2].wait()
    copies_out[-1].wait()


_sel = pl.kernel(
    _sel_body,
    out_type=jax.ShapeDtypeStruct((_OUT_TOTAL,), jnp.float32),
    mesh=plsc.VectorSubcoreMesh(
        core_axis_name="c",
        subcore_axis_name="s",
        num_cores=_NUM_CORES,
        num_subcores=_NUM_SUBCORES,
    ),
    scratch_types=[
        pltpu.VMEM((_IN_CHUNK,), jnp.float32),
        pltpu.VMEM((_IN_CHUNK,), jnp.float32),
        pltpu.VMEM((_OUT_CHUNK,), jnp.float32),
        pltpu.VMEM((_OUT_CHUNK,), jnp.float32),
        pltpu.SemaphoreType.DMA,
        pltpu.SemaphoreType.DMA,
    ],
    compiler_params=pltpu.CompilerParams(needs_layout_passes=False),
)


def kernel(firings):
    out = _sel(firings.reshape(_IN_TOTAL))
    return out.reshape(_B, _R, _OF)


# speedup vs baseline: 1.6771x; 1.4116x over previous
import jax
import jax.numpy as jnp
from jax import lax
from jax.experimental import pallas as pl
from jax.experimental.pallas import tpu as pltpu
from jax.experimental.pallas import tpu_sc as plsc

_LANES = 16
_NUM_CORES = 2
_NUM_SUBCORES = 16
_NW = _NUM_CORES * _NUM_SUBCORES

_B, _R, _F = 4, 4096, 256
_OF = _F // 2
_ROWS = _B * _R
_ROWS_PER_W = _ROWS // _NW
_NCHUNK = 4
_CROWS = _ROWS_PER_W // _NCHUNK
_VPR = _OF // _LANES


def _sel_body(x_hbm, out_hbm, in_v0, in_v1, out_v0, out_v1, in_sem, out_sem):
    in_bufs = (in_v0, in_v1)
    out_bufs = (out_v0, out_v1)
    wid = lax.axis_index("s") * _NUM_CORES + lax.axis_index("c")
    b = wid // (_R // _ROWS_PER_W)
    r0 = (wid % (_R // _ROWS_PER_W)) * _ROWS_PER_W

    iota = lax.iota(jnp.int32, _LANES)
    cols = [iota * 2 + 32 * vj for vj in range(_VPR)]

    def start_in(t):
        return pltpu.async_copy(
            x_hbm.at[b, pl.ds(r0 + t * _CROWS, _CROWS), :],
            in_bufs[t % 2], in_sem)

    def start_out(t):
        return pltpu.async_copy(
            out_bufs[t % 2],
            out_hbm.at[b, pl.ds(r0 + t * _CROWS, _CROWS), :], out_sem)

    copies_in = [start_in(0)]
    copies_out = []
    for t in range(_NCHUNK):
        if t + 1 < _NCHUNK:
            copies_in.append(start_in(t + 1))
        copies_in[t].wait()
        if t >= 2:
            copies_out[t - 2].wait()
        src = in_bufs[t % 2]
        dst = out_bufs[t % 2]

        def body(dr, carry):
            row = iota * 0 + dr
            for vj in range(_VPR):
                dst[dr, pl.ds(vj * _LANES, _LANES)] = (
                    plsc.load_gather(src, [row, cols[vj]]))
            return carry

        lax.fori_loop(0, _CROWS, body, 0)
        copies_out.append(start_out(t))
    copies_out[-2].wait()
    copies_out[-1].wait()


_sel = pl.kernel(
    _sel_body,
    out_type=jax.ShapeDtypeStruct((_B, _R, _OF), jnp.float32),
    mesh=plsc.VectorSubcoreMesh(
        core_axis_name="c",
        subcore_axis_name="s",
        num_cores=_NUM_CORES,
        num_subcores=_NUM_SUBCORES,
    ),
    scratch_types=[
        pltpu.VMEM((_CROWS, _F), jnp.float32),
        pltpu.VMEM((_CROWS, _F), jnp.float32),
        pltpu.VMEM((_CROWS, _OF), jnp.float32),
        pltpu.VMEM((_CROWS, _OF), jnp.float32),
        pltpu.SemaphoreType.DMA,
        pltpu.SemaphoreType.DMA,
    ],
    compiler_params=pltpu.CompilerParams(needs_layout_passes=False),
)


def kernel(firings):
    return _sel(firings)
